# baseline (speedup 1.0000x reference)
import functools

import jax
import jax.numpy as jnp
from jax import lax
from jax.experimental import pallas as pl
from jax.experimental.pallas import tpu as pltpu
from jax.experimental.pallas import tpu_sc as plsc

BATCH = 16384
EMB = 64

_info = plsc.get_sparse_core_info()
_NC, _NS = _info.num_cores, _info.num_subcores
_NW = _NC * _NS
_BPW = BATCH // _NW


def _gather_body(user_hbm, item_hbm, ug_hbm, ig_hbm, um_hbm, im_hbm,
                 eu_out, ei_out, mu_out, mi_out,
                 idx_u, idx_i, buf0, buf1, sem0, sem1):
    wid = lax.axis_index("s") * _NC + lax.axis_index("c")
    base = wid * _BPW
    pltpu.sync_copy(user_hbm.at[pl.ds(base, _BPW)], idx_u)
    pltpu.sync_copy(item_hbm.at[pl.ds(base, _BPW)], idx_i)

    cp0 = pltpu.async_copy(ug_hbm.at[idx_u], buf0, sem0)
    cp1 = pltpu.async_copy(ig_hbm.at[idx_i], buf1, sem1)
    cp0.wait()
    pltpu.sync_copy(buf0, eu_out.at[pl.ds(base, _BPW)])
    cp1.wait()
    pltpu.sync_copy(buf1, ei_out.at[pl.ds(base, _BPW)])

    cp0 = pltpu.async_copy(um_hbm.at[idx_u], buf0, sem0)
    cp1 = pltpu.async_copy(im_hbm.at[idx_i], buf1, sem1)
    cp0.wait()
    pltpu.sync_copy(buf0, mu_out.at[pl.ds(base, _BPW)])
    cp1.wait()
    pltpu.sync_copy(buf1, mi_out.at[pl.ds(base, _BPW)])


_row_ty = jax.ShapeDtypeStruct((BATCH, EMB), jnp.float32)

_sc_gather = pl.kernel(
    _gather_body,
    out_type=(_row_ty, _row_ty, _row_ty, _row_ty),
    mesh=plsc.VectorSubcoreMesh(core_axis_name="c", subcore_axis_name="s"),
    scratch_types=[
        pltpu.VMEM((_BPW,), jnp.int32),
        pltpu.VMEM((_BPW,), jnp.int32),
        pltpu.VMEM((_BPW, EMB), jnp.float32),
        pltpu.VMEM((_BPW, EMB), jnp.float32),
        pltpu.SemaphoreType.DMA,
        pltpu.SemaphoreType.DMA,
    ],
    compiler_params=pltpu.CompilerParams(use_tc_tiling_on_sc=False),
)


_BB = 2048


def _mlp_body(eu, ei, mu, mi, w1a, w1b, b1, w2, b2, w3, b3, wg, wh, bp, out):
    cdims = (((1,), (1,)), ((), ()))
    hp = jax.lax.Precision.HIGHEST
    gmf = eu[...] * ei[...]
    h = lax.dot_general(mu[...], w1a[...], cdims, precision=hp)
    h = h + lax.dot_general(mi[...], w1b[...], cdims, precision=hp)
    h = jnp.maximum(h + b1[...], 0.0)
    h = jnp.maximum(lax.dot_general(h, w2[...], cdims, precision=hp) + b2[...], 0.0)
    h = jnp.maximum(lax.dot_general(h, w3[...], cdims, precision=hp) + b3[...], 0.0)
    pred = jnp.sum(gmf * wg[...], axis=1) + jnp.sum(h * wh[...], axis=1)
    out[...] = pred + bp[0]


def _full(shape):
    nd = len(shape)
    return pl.BlockSpec(shape, lambda i: (0,) * nd)


def kernel(user, item, Ug, Ig, Um, Im, W1, b1, W2, b2, W3, b3, Wp, bp):
    user = user.astype(jnp.int32)
    item = item.astype(jnp.int32)
    eu, ei, mu, mi = _sc_gather(user, item, Ug, Ig, Um, Im)

    w1a = W1[:, :EMB]
    w1b = W1[:, EMB:]
    wg = Wp[:, :EMB]
    wh = Wp[:, EMB:]
    h1 = W1.shape[0]
    h2 = W2.shape[0]
    h3 = W3.shape[0]

    grid = BATCH // _BB
    row_spec = pl.BlockSpec((_BB, EMB), lambda i: (i, 0))
    out = pl.pallas_call(
        _mlp_body,
        grid=(grid,),
        in_specs=[
            row_spec, row_spec, row_spec, row_spec,
            _full((h1, EMB)), _full((h1, EMB)), _full((1, h1)),
            _full((h2, h1)), _full((1, h2)),
            _full((h3, h2)), _full((1, h3)),
            _full((1, EMB)), _full((1, h3)), _full((1,)),
        ],
        out_specs=pl.BlockSpec((_BB,), lambda i: (i,)),
        out_shape=jax.ShapeDtypeStruct((BATCH,), jnp.float32),
    )(eu, ei, mu, mi, w1a, w1b, b1.reshape(1, h1), W2, b2.reshape(1, h2),
      W3, b3.reshape(1, h3), wg, wh, bp)
    return out
